# baseline (speedup 1.0000x reference)
import functools
import math

import jax
import jax.numpy as jnp
import numpy as np
from jax import lax
from jax.experimental import pallas as pl
from jax.experimental.pallas import tpu as pltpu
from jax.experimental.pallas import tpu_sc as plsc

D_MODEL = 512
MAX_LEN = 512

NUM_SC = 2
NUM_SUBCORES = 16
NW = NUM_SC * NUM_SUBCORES


def _pe_table_ext() -> np.ndarray:
    pe = np.zeros((MAX_LEN, D_MODEL), dtype=np.float32)
    position = np.arange(0, MAX_LEN, dtype=np.float32)[:, None]
    div_term = np.exp(
        np.arange(0, D_MODEL, 2, dtype=np.float32) * -(math.log(10000.0) / D_MODEL)
    )
    pe[:, 0::2] = np.sin(position * div_term)
    pe[:, 1::2] = np.cos(position * div_term)
    return np.concatenate([np.zeros((1, D_MODEL), np.float32), pe], axis=0)


def _gather_indices(t_lens, D, n_pad: int) -> np.ndarray:
    parts = []
    for t in t_lens:
        parts.append(np.zeros((1,), np.int32))
        parts.append(np.linspace(0, D - 1, t).astype(np.int32) + 1)
    idx = np.concatenate(parts)
    return np.concatenate([idx, np.zeros((n_pad - idx.shape[0],), np.int32)])


def _sc_gather(table: jax.Array, idx: jax.Array, n_pad: int) -> jax.Array:
    b_per_w = n_pad // NW
    mesh = plsc.VectorSubcoreMesh(
        core_axis_name="c", subcore_axis_name="s",
        num_cores=NUM_SC, num_subcores=NUM_SUBCORES,
    )

    @functools.partial(
        pl.kernel,
        out_type=jax.ShapeDtypeStruct((n_pad, D_MODEL), jnp.float32),
        mesh=mesh,
        scratch_types=[
            pltpu.VMEM((b_per_w,), jnp.int32),
            pltpu.VMEM((b_per_w, D_MODEL), jnp.float32),
            pltpu.SemaphoreType.DMA,
        ],
    )
    def gather_kernel(table_hbm, idx_hbm, out_hbm, idx_v, rows_v, sem):
        wid = lax.axis_index("s") * NUM_SC + lax.axis_index("c")
        base = wid * b_per_w
        pltpu.sync_copy(idx_hbm.at[pl.ds(base, b_per_w)], idx_v)
        pltpu.async_copy(table_hbm.at[idx_v], rows_v, sem).wait()
        pltpu.sync_copy(rows_v, out_hbm.at[pl.ds(base, b_per_w)])

    return gather_kernel(table, idx)


def _tc_broadcast(temp: jax.Array, batch: int) -> jax.Array:
    seq, d = temp.shape

    def bcast_kernel(t_ref, o_ref, sem):
        for b in range(batch):
            pltpu.make_async_copy(t_ref, o_ref.at[b], sem).start()
        for b in range(batch):
            pltpu.make_async_copy(t_ref, o_ref.at[b], sem).wait()

    return pl.pallas_call(
        bcast_kernel,
        in_specs=[pl.BlockSpec((seq, d), lambda: (0, 0))],
        out_specs=pl.BlockSpec(memory_space=pl.ANY),
        out_shape=jax.ShapeDtypeStruct((batch, seq, d), jnp.float32),
        scratch_shapes=[pltpu.SemaphoreType.DMA],
    )(temp)


def kernel(modal_feat_0, modal_feat_1, modal_feat_2):
    modal_feats = (modal_feat_0, modal_feat_1, modal_feat_2)
    batch = modal_feats[0].shape[0]
    D = modal_feats[0].shape[1] - 1
    t_lens = [m.shape[1] - 1 for m in modal_feats]
    seq = sum(t_lens) + len(t_lens)

    n_pad = -(-seq // (8 * NW)) * (8 * NW)
    table = jnp.asarray(_pe_table_ext())
    idx = jnp.asarray(_gather_indices(t_lens, D, n_pad))

    temp = _sc_gather(table, idx, n_pad)
    return _tc_broadcast(temp[:seq], batch)


# speedup vs baseline: 1.3332x; 1.3332x over previous
import functools
import math

import jax
import jax.numpy as jnp
import numpy as np
from jax import lax
from jax.experimental import pallas as pl
from jax.experimental.pallas import tpu as pltpu

D_MODEL = 512
MAX_LEN = 512


def _pe_table_ext() -> np.ndarray:
    pe = np.zeros((MAX_LEN, D_MODEL), dtype=np.float32)
    position = np.arange(0, MAX_LEN, dtype=np.float32)[:, None]
    div_term = np.exp(
        np.arange(0, D_MODEL, 2, dtype=np.float32) * -(math.log(10000.0) / D_MODEL)
    )
    pe[:, 0::2] = np.sin(position * div_term)
    pe[:, 1::2] = np.cos(position * div_term)
    return np.concatenate([np.zeros((1, D_MODEL), np.float32), pe], axis=0)


def _gather_indices(t_lens, D) -> np.ndarray:
    parts = []
    for t in t_lens:
        parts.append(np.zeros((1,), np.int32))
        parts.append(np.linspace(0, D - 1, t).astype(np.int32) + 1)
    return np.concatenate(parts)


def kernel(modal_feat_0, modal_feat_1, modal_feat_2):
    modal_feats = (modal_feat_0, modal_feat_1, modal_feat_2)
    batch = modal_feats[0].shape[0]
    D = modal_feats[0].shape[1] - 1
    t_lens = [m.shape[1] - 1 for m in modal_feats]
    seq = sum(t_lens) + len(t_lens)

    table = _pe_table_ext()
    idx = _gather_indices(t_lens, D)
    nrows = table.shape[0]
    onehot = np.zeros((seq, nrows), np.float32)
    onehot[np.arange(seq), idx] = 1.0

    def body(oh_ref, tab_ref, o_ref, temp, sem):
        temp[...] = jnp.dot(
            oh_ref[...], tab_ref[...], preferred_element_type=jnp.float32
        )
        for b in range(batch):
            pltpu.make_async_copy(temp, o_ref.at[b], sem).start()
        for b in range(batch):
            pltpu.make_async_copy(temp, o_ref.at[b], sem).wait()

    return pl.pallas_call(
        body,
        in_specs=[
            pl.BlockSpec((seq, nrows), lambda: (0, 0)),
            pl.BlockSpec((nrows, D_MODEL), lambda: (0, 0)),
        ],
        out_specs=pl.BlockSpec(memory_space=pl.ANY),
        out_shape=jax.ShapeDtypeStruct((batch, seq, D_MODEL), jnp.float32),
        scratch_shapes=[
            pltpu.VMEM((seq, D_MODEL), jnp.float32),
            pltpu.SemaphoreType.DMA,
        ],
    )(jnp.asarray(onehot), jnp.asarray(table))
